# baseline (speedup 1.0000x reference)
import functools

import jax
import jax.numpy as jnp
import numpy as np
from jax import lax
from jax.experimental import pallas as pl
from jax.experimental.pallas import tpu as pltpu
from jax.experimental.pallas import tpu_sc as plsc

_N_BINS = 15
_BLOCK = 32768
_SC_SAMPLES = 294912
_SC_W = 512
_NW = 32
_BOUNDS = np.array(
    [0x0, 0x3D888889, 0x3E088889, 0x3E4CCCCE, 0x3E888889, 0x3EAAAAAB,
     0x3ECCCCCE, 0x3EEEEEF0, 0x3F088889, 0x3F19999A, 0x3F2AAAAB,
     0x3F3BBBBC, 0x3F4CCCCE, 0x3F5DDDDF, 0x3F6EEEF0, 0x3F800000],
    dtype=np.uint32).view(np.float32)


def _tc_kernel(start, n_all, x_ref, lab_ref, out_ref, cnt_ref, sc_ref,
               sa_ref):
    i = pl.program_id(0)
    nb = pl.num_programs(0)

    @pl.when(i == 0)
    def _init():
        cnt_ref[...] = jnp.zeros_like(cnt_ref)
        sc_ref[...] = jnp.zeros_like(sc_ref)
        sa_ref[...] = jnp.zeros_like(sa_ref)
        out_ref[...] = jnp.zeros_like(out_ref)

    x = x_ref[...]
    lab = lab_ref[0]
    c_iota = jax.lax.broadcasted_iota(jnp.int32, x.shape, 0)
    conf = jnp.max(x, axis=0, keepdims=True)
    vlab = jnp.max(jnp.where(c_iota == lab, x, -jnp.inf), axis=0,
                   keepdims=True)
    acc = (vlab == conf).astype(jnp.float32)

    sub = _BLOCK // 8
    conf8 = conf.reshape(8, sub)
    acc8 = acc.reshape(8, sub)
    idx = (start + i * _BLOCK
           + jax.lax.broadcasted_iota(jnp.int32, (8, sub), 0) * sub
           + jax.lax.broadcasted_iota(jnp.int32, (8, sub), 1))
    conf8 = jnp.where(idx < jnp.int32(n_all), conf8, -1.0)

    for j in range(_N_BINS + 1):
        m = (conf8 > _BOUNDS[j]).astype(jnp.float32)
        cnt_ref[j] += m
        sc_ref[j] += conf8 * m
        sa_ref[j] += acc8 * m

    @pl.when(i == nb - 1)
    def _final():
        tcnt = jnp.sum(cnt_ref[...], axis=(1, 2))
        tsc = jnp.sum(sc_ref[...], axis=(1, 2))
        tsa = jnp.sum(sa_ref[...], axis=(1, 2))
        out_ref[...] = jnp.concatenate([tcnt, tsc, tsa]).reshape(1, 1, 48)


def _make_sc_kernel(s0, per_w, n_chunks):
    mesh = plsc.VectorSubcoreMesh(core_axis_name="c", subcore_axis_name="s")

    @functools.partial(
        pl.kernel,
        out_type=jax.ShapeDtypeStruct((_NW, 1, 48), jnp.float32),
        mesh=mesh,
        scratch_types=[
            pltpu.VMEM((100, _SC_W), jnp.float32),
            pltpu.VMEM((_SC_W,), jnp.int32),
            pltpu.VMEM((16, 16), jnp.float32),
            pltpu.VMEM((16, 16), jnp.float32),
            pltpu.VMEM((16, 16), jnp.float32),
            pltpu.VMEM((48,), jnp.float32),
        ],
        compiler_params=pltpu.CompilerParams(needs_layout_passes=False),
    )
    def _sc_kernel(x_hbm, lab_hbm, out_hbm, xbuf, labbuf,
                   cnt_tab, sct_tab, sat_tab, outvec):
        wid = lax.axis_index("s") * 2 + lax.axis_index("c")
        base = s0 + wid * per_w
        zero16 = jnp.zeros((16,), jnp.float32)
        for r in range(16):
            cnt_tab[r] = zero16
            sct_tab[r] = zero16
            sat_tab[r] = zero16
        lane = lax.iota(jnp.int32, 16)
        ones = jnp.ones((16,), jnp.float32)

        def chunk_body(k, carry):
            off = base + k * _SC_W
            pltpu.sync_copy(x_hbm.at[:, pl.ds(off, _SC_W)], xbuf)
            pltpu.sync_copy(lab_hbm.at[pl.ds(off, _SC_W)], labbuf)

            def group_body(g, c2):
                sl = pl.ds(g * 16, 16)
                conf = xbuf[0, sl]
                for cc in range(1, 100):
                    conf = jnp.maximum(conf, xbuf[cc, sl])
                labg = labbuf[sl]
                vlab = plsc.load_gather(xbuf, [labg, g * 16 + lane])
                accf = jnp.where(vlab == conf, ones, 0.0)
                b = jnp.full((16,), -1, jnp.int32)
                for j in range(16):
                    b = b + jnp.where(conf > _BOUNDS[j], 1, 0)
                msk = (b >= 0) & (b <= 14)
                bsafe = jnp.clip(b, 0, 15)
                plsc.addupdate_scatter(cnt_tab, [bsafe, lane], ones, mask=msk)
                plsc.addupdate_scatter(sct_tab, [bsafe, lane], conf, mask=msk)
                plsc.addupdate_scatter(sat_tab, [bsafe, lane], accf, mask=msk)
                return c2

            return lax.fori_loop(0, _SC_W // 16, group_body, carry)

        lax.fori_loop(0, n_chunks, chunk_body, 0)

        def _rowsums(tab):
            tot = jnp.zeros((16,), jnp.float32)
            for col in range(16):
                tot = tot + plsc.load_gather(
                    tab, [lane, jnp.full((16,), col, jnp.int32)])
            return tot

        outvec[pl.ds(0, 16)] = _rowsums(cnt_tab)
        outvec[pl.ds(16, 16)] = _rowsums(sct_tab)
        outvec[pl.ds(32, 16)] = _rowsums(sat_tab)
        pltpu.sync_copy(outvec, out_hbm.at[wid, 0])

    return _sc_kernel


def _merge_kernel(n_total, tc_ref, sc_ref, out_ref):
    tc = tc_ref[0, 0]
    scp = jnp.sum(sc_ref[...], axis=(0, 1))
    tcnt = tc[:16]
    tsc = tc[16:32]
    tsa = tc[32:]
    cnt = tcnt[:_N_BINS] - tcnt[1:] + scp[:_N_BINS]
    sconf = tsc[:_N_BINS] - tsc[1:] + scp[16:16 + _N_BINS]
    sacc = tsa[:_N_BINS] - tsa[1:] + scp[32:32 + _N_BINS]
    safe = jnp.maximum(cnt, 1.0)
    contrib = jnp.abs(sconf / safe - sacc / safe) * (cnt / n_total)
    contrib = jnp.where(cnt > 0.0, contrib, 0.0)
    out_ref[...] = jnp.sum(contrib).reshape(1, 1)


def kernel(softmaxes, labels):
    n, c = softmaxes.shape
    n_sc = _SC_SAMPLES
    n_tc = n - n_sc
    sc_blocks = n_sc // _BLOCK
    nb = (n_tc + _BLOCK - 1) // _BLOCK
    xt = softmaxes.T
    labels32 = labels.astype(jnp.int32)
    lab_pad = jnp.pad(labels32[n_sc:], (0, nb * _BLOCK - n_tc))
    lab3 = lab_pad.reshape(nb, 1, _BLOCK)
    tc_parts = pl.pallas_call(
        functools.partial(_tc_kernel, n_sc, float(n)),
        grid=(nb,),
        in_specs=[
            pl.BlockSpec((c, _BLOCK), lambda i: (0, i + sc_blocks)),
            pl.BlockSpec((1, 1, _BLOCK), lambda i: (i, 0, 0)),
        ],
        out_specs=pl.BlockSpec((1, 1, 48), lambda i: (0, 0, 0)),
        out_shape=jax.ShapeDtypeStruct((1, 1, 48), jnp.float32),
        scratch_shapes=[
            pltpu.VMEM((_N_BINS + 1, 8, _BLOCK // 8), jnp.float32),
            pltpu.VMEM((_N_BINS + 1, 8, _BLOCK // 8), jnp.float32),
            pltpu.VMEM((_N_BINS + 1, 8, _BLOCK // 8), jnp.float32),
        ],
        compiler_params=pltpu.CompilerParams(
            dimension_semantics=("arbitrary",),
        ),
    )(xt, lab3)
    per_w = n_sc // _NW
    sc_parts = _make_sc_kernel(0, per_w, per_w // _SC_W)(xt, labels32)
    out = pl.pallas_call(
        functools.partial(_merge_kernel, float(n)),
        in_specs=[
            pl.BlockSpec((1, 1, 48), lambda: (0, 0, 0)),
            pl.BlockSpec((_NW, 1, 48), lambda: (0, 0, 0)),
        ],
        out_specs=pl.BlockSpec((1, 1), lambda: (0, 0)),
        out_shape=jax.ShapeDtypeStruct((1, 1), softmaxes.dtype),
    )(tc_parts, sc_parts)
    return out.reshape(1)


# speedup vs baseline: 1.0053x; 1.0053x over previous
import functools

import jax
import jax.numpy as jnp
import numpy as np
from jax import lax
from jax.experimental import pallas as pl
from jax.experimental.pallas import tpu as pltpu
from jax.experimental.pallas import tpu_sc as plsc

_N_BINS = 15
_BLOCK = 32768
_SC_SAMPLES = 262144
_SC_W = 512
_NW = 32
_BOUNDS = np.array(
    [0x0, 0x3D888889, 0x3E088889, 0x3E4CCCCE, 0x3E888889, 0x3EAAAAAB,
     0x3ECCCCCE, 0x3EEEEEF0, 0x3F088889, 0x3F19999A, 0x3F2AAAAB,
     0x3F3BBBBC, 0x3F4CCCCE, 0x3F5DDDDF, 0x3F6EEEF0, 0x3F800000],
    dtype=np.uint32).view(np.float32)


def _tc_kernel(start, n_all, x_ref, lab_ref, out_ref, cnt_ref, sc_ref,
               sa_ref):
    i = pl.program_id(0)
    nb = pl.num_programs(0)

    @pl.when(i == 0)
    def _init():
        cnt_ref[...] = jnp.zeros_like(cnt_ref)
        sc_ref[...] = jnp.zeros_like(sc_ref)
        sa_ref[...] = jnp.zeros_like(sa_ref)
        out_ref[...] = jnp.zeros_like(out_ref)

    x = x_ref[...]
    lab = lab_ref[0]
    c_iota = jax.lax.broadcasted_iota(jnp.int32, x.shape, 0)
    conf = jnp.max(x, axis=0, keepdims=True)
    vlab = jnp.max(jnp.where(c_iota == lab, x, -jnp.inf), axis=0,
                   keepdims=True)
    acc = (vlab == conf).astype(jnp.float32)

    sub = _BLOCK // 8
    conf8 = conf.reshape(8, sub)
    acc8 = acc.reshape(8, sub)
    idx = (start + i * _BLOCK
           + jax.lax.broadcasted_iota(jnp.int32, (8, sub), 0) * sub
           + jax.lax.broadcasted_iota(jnp.int32, (8, sub), 1))
    conf8 = jnp.where(idx < jnp.int32(n_all), conf8, -1.0)

    for j in range(_N_BINS + 1):
        m = (conf8 > _BOUNDS[j]).astype(jnp.float32)
        cnt_ref[j] += m
        sc_ref[j] += conf8 * m
        sa_ref[j] += acc8 * m

    @pl.when(i == nb - 1)
    def _final():
        tcnt = jnp.sum(cnt_ref[...], axis=(1, 2))
        tsc = jnp.sum(sc_ref[...], axis=(1, 2))
        tsa = jnp.sum(sa_ref[...], axis=(1, 2))
        out_ref[...] = jnp.concatenate([tcnt, tsc, tsa]).reshape(1, 1, 48)


def _make_sc_kernel(s0, per_w, n_chunks):
    mesh = plsc.VectorSubcoreMesh(core_axis_name="c", subcore_axis_name="s")

    @functools.partial(
        pl.kernel,
        out_type=jax.ShapeDtypeStruct((_NW, 1, 48), jnp.float32),
        mesh=mesh,
        scratch_types=[
            pltpu.VMEM((100, _SC_W), jnp.float32),
            pltpu.VMEM((_SC_W,), jnp.int32),
            pltpu.VMEM((16, 16), jnp.float32),
            pltpu.VMEM((16, 16), jnp.float32),
            pltpu.VMEM((16, 16), jnp.float32),
            pltpu.VMEM((48,), jnp.float32),
        ],
        compiler_params=pltpu.CompilerParams(needs_layout_passes=False),
    )
    def _sc_kernel(x_hbm, lab_hbm, out_hbm, xbuf, labbuf,
                   cnt_tab, sct_tab, sat_tab, outvec):
        wid = lax.axis_index("s") * 2 + lax.axis_index("c")
        base = s0 + wid * per_w
        zero16 = jnp.zeros((16,), jnp.float32)
        for r in range(16):
            cnt_tab[r] = zero16
            sct_tab[r] = zero16
            sat_tab[r] = zero16
        lane = lax.iota(jnp.int32, 16)
        ones = jnp.ones((16,), jnp.float32)

        def chunk_body(k, carry):
            off = base + k * _SC_W
            pltpu.sync_copy(x_hbm.at[:, pl.ds(off, _SC_W)], xbuf)
            pltpu.sync_copy(lab_hbm.at[pl.ds(off, _SC_W)], labbuf)

            def group_body(g, c2):
                sl = pl.ds(g * 16, 16)
                conf = xbuf[0, sl]
                for cc in range(1, 100):
                    conf = jnp.maximum(conf, xbuf[cc, sl])
                labg = labbuf[sl]
                vlab = plsc.load_gather(xbuf, [labg, g * 16 + lane])
                accf = jnp.where(vlab == conf, ones, 0.0)
                b = jnp.full((16,), -1, jnp.int32)
                for j in range(16):
                    b = b + jnp.where(conf > _BOUNDS[j], 1, 0)
                msk = (b >= 0) & (b <= 14)
                bsafe = jnp.clip(b, 0, 15)
                plsc.addupdate_scatter(cnt_tab, [bsafe, lane], ones, mask=msk)
                plsc.addupdate_scatter(sct_tab, [bsafe, lane], conf, mask=msk)
                plsc.addupdate_scatter(sat_tab, [bsafe, lane], accf, mask=msk)
                return c2

            return lax.fori_loop(0, _SC_W // 16, group_body, carry)

        lax.fori_loop(0, n_chunks, chunk_body, 0)

        def _rowsums(tab):
            tot = jnp.zeros((16,), jnp.float32)
            for col in range(16):
                tot = tot + plsc.load_gather(
                    tab, [lane, jnp.full((16,), col, jnp.int32)])
            return tot

        outvec[pl.ds(0, 16)] = _rowsums(cnt_tab)
        outvec[pl.ds(16, 16)] = _rowsums(sct_tab)
        outvec[pl.ds(32, 16)] = _rowsums(sat_tab)
        pltpu.sync_copy(outvec, out_hbm.at[wid, 0])

    return _sc_kernel


def _merge_kernel(n_total, tc_ref, sc_ref, out_ref):
    tc = tc_ref[0, 0]
    scp = jnp.sum(sc_ref[...], axis=(0, 1))
    tcnt = tc[:16]
    tsc = tc[16:32]
    tsa = tc[32:]
    cnt = tcnt[:_N_BINS] - tcnt[1:] + scp[:_N_BINS]
    sconf = tsc[:_N_BINS] - tsc[1:] + scp[16:16 + _N_BINS]
    sacc = tsa[:_N_BINS] - tsa[1:] + scp[32:32 + _N_BINS]
    safe = jnp.maximum(cnt, 1.0)
    contrib = jnp.abs(sconf / safe - sacc / safe) * (cnt / n_total)
    contrib = jnp.where(cnt > 0.0, contrib, 0.0)
    out_ref[...] = jnp.sum(contrib).reshape(1, 1)


def kernel(softmaxes, labels):
    n, c = softmaxes.shape
    n_sc = _SC_SAMPLES
    n_tc = n - n_sc
    sc_blocks = n_sc // _BLOCK
    nb = (n_tc + _BLOCK - 1) // _BLOCK
    xt = softmaxes.T
    labels32 = labels.astype(jnp.int32)
    lab_pad = jnp.pad(labels32[n_sc:], (0, nb * _BLOCK - n_tc))
    lab3 = lab_pad.reshape(nb, 1, _BLOCK)
    tc_parts = pl.pallas_call(
        functools.partial(_tc_kernel, n_sc, float(n)),
        grid=(nb,),
        in_specs=[
            pl.BlockSpec((c, _BLOCK), lambda i: (0, i + sc_blocks)),
            pl.BlockSpec((1, 1, _BLOCK), lambda i: (i, 0, 0)),
        ],
        out_specs=pl.BlockSpec((1, 1, 48), lambda i: (0, 0, 0)),
        out_shape=jax.ShapeDtypeStruct((1, 1, 48), jnp.float32),
        scratch_shapes=[
            pltpu.VMEM((_N_BINS + 1, 8, _BLOCK // 8), jnp.float32),
            pltpu.VMEM((_N_BINS + 1, 8, _BLOCK // 8), jnp.float32),
            pltpu.VMEM((_N_BINS + 1, 8, _BLOCK // 8), jnp.float32),
        ],
        compiler_params=pltpu.CompilerParams(
            dimension_semantics=("arbitrary",),
        ),
    )(xt, lab3)
    per_w = n_sc // _NW
    sc_parts = _make_sc_kernel(0, per_w, per_w // _SC_W)(xt, labels32)
    out = pl.pallas_call(
        functools.partial(_merge_kernel, float(n)),
        in_specs=[
            pl.BlockSpec((1, 1, 48), lambda: (0, 0, 0)),
            pl.BlockSpec((_NW, 1, 48), lambda: (0, 0, 0)),
        ],
        out_specs=pl.BlockSpec((1, 1), lambda: (0, 0)),
        out_shape=jax.ShapeDtypeStruct((1, 1), softmaxes.dtype),
    )(tc_parts, sc_parts)
    return out.reshape(1)
